# baseline (speedup 1.0000x reference)
import functools

import jax
import jax.numpy as jnp
import numpy as np
from jax import lax
from jax.experimental import pallas as pl
from jax.experimental.pallas import tpu as pltpu
from jax.experimental.pallas import tpu_sc as plsc

N = 10000
E = 320000
D_IN = 128
LAT = 32
MAX_LOGSTD = 10.0

HS_W = 160
AC_W = 144
AD_W = 16
NC = 2
NS = 16
NW = NC * NS
CHUNK = 40
NCHT = 256
E_PAD = NW * NCHT * CHUNK
N_PAD = 10240
RPT = N_PAD // NS

_ROWBLK = 1000
_FROWBLK = 1024

_PERM = np.empty(D_IN, np.int32)
for _j in range(4):
    for _k in range(16):
        _PERM[32 * _j + 2 * _k] = 32 * _j + _k
        _PERM[32 * _j + 2 * _k + 1] = 32 * _j + 16 + _k


def _proj_body(x_ref, w_ref, amat_s_ref, amat_d_ref, hs_ref, ad_ref):
    h = jnp.dot(x_ref[...], w_ref[...], preferred_element_type=jnp.float32)
    hs_ref[:, 0:D_IN] = h.astype(jnp.bfloat16)
    hs_ref[:, D_IN:D_IN + 16] = jnp.dot(
        h, amat_s_ref[...],
        preferred_element_type=jnp.float32).astype(jnp.bfloat16)
    hs_ref[:, D_IN + 16:HS_W] = jnp.zeros(
        (_ROWBLK, 16), jnp.bfloat16)
    ad_ref[...] = jnp.dot(h, amat_d_ref[...],
                          preferred_element_type=jnp.float32)


def _finalize_body(p_ref, noise_ref, z_ref):
    a = p_ref[0] + p_ref[1]
    eps = 1e-16
    mu1 = a[:, 0:32] / (a[:, 128:129] + eps)
    ls1 = a[:, 32:64] / (a[:, 129:130] + eps)
    mu2 = a[:, 64:96] / (a[:, 130:131] + eps)
    ls2 = a[:, 96:128] / (a[:, 131:132] + eps)
    z_ref[:, 0:32] = mu1 + noise_ref[:, 0:32] * jnp.exp(
        jnp.minimum(ls1, MAX_LOGSTD))
    z_ref[:, 32:64] = mu2 + noise_ref[:, 32:64] * jnp.exp(
        jnp.minimum(ls2, MAX_LOGSTD))


def _edge_body(pk_hbm, hs_hbm, ad_hbm, out_hbm,
               pk, srcv, dstv, rows, adr, scat, acc, isem, gsem, ssem):
    c = lax.axis_index("c")
    s = lax.axis_index("s")
    wid = s * NC + c

    pltpu.async_copy(pk_hbm.at[wid], pk, isem)

    zb = scat[0]
    for r in range(CHUNK):
        for o in range(0, AC_W, 16):
            zb[r, pl.ds(o, 16)] = jnp.zeros((16,), jnp.float32)
    for t in range(RPT // CHUNK):
        pltpu.sync_copy(zb, acc.at[pl.ds(s * RPT + t * CHUNK, CHUNK)])

    pltpu.make_async_copy(pk_hbm.at[wid], pk, isem).wait()
    plsc.subcore_barrier()

    def unpack_idx(b, j):
        for o in (0, 16, 24):
            v = pk[j, pl.ds(o, 16)]
            srcv[b][pl.ds(o, 16)] = lax.shift_right_logical(v, 14)
            dstv[b][pl.ds(o, 16)] = jnp.bitwise_and(v, 16383)

    def fire_gather(b):
        pltpu.async_copy(hs_hbm.at[srcv[b]], rows[b], gsem[b])
        pltpu.async_copy(ad_hbm.at[dstv[b]], adr[b], gsem[b])

    def wait_gather(b):
        pltpu.make_async_copy(hs_hbm.at[srcv[b]], rows[b], gsem[b]).wait()
        pltpu.make_async_copy(ad_hbm.at[dstv[b]], adr[b], gsem[b]).wait()

    def fire_scatter(sb, b):
        pltpu.async_copy(scat[sb], acc.at[dstv[b]], ssem[sb], add=True)

    def wait_scatter(sb):
        pltpu.make_async_copy(scat[sb], acc.at[dstv[sb]], ssem[sb]).wait()

    def compute(b, sb):
        rv = rows[b]
        ar = adr[b]
        sc_ = scat[sb]

        def unpack_bf16(vi):
            lo = lax.bitcast_convert_type(lax.shift_left(vi, 16),
                                          jnp.float32)
            hi = lax.bitcast_convert_type(
                jnp.bitwise_and(vi, jnp.int32(-65536)), jnp.float32)
            return lo, hi

        def edge_body(e, carry):
            asr, _ = unpack_bf16(rv[e, pl.ds(64, 16)])
            av = asr + ar[e, :]
            ev = jnp.where(av > 0.0, av, av * jnp.float32(0.2))
            exv = jnp.exp(ev)
            sc_[e, pl.ds(D_IN, 16)] = exv
            for j in range(4):
                bc = lax.gather(
                    exv, jnp.full((16, 1), j, jnp.int32),
                    lax.GatherDimensionNumbers(
                        offset_dims=(), collapsed_slice_dims=(0,),
                        start_index_map=(0,)),
                    slice_sizes=(1,),
                    mode=lax.GatherScatterMode.PROMISE_IN_BOUNDS)
                lo, hi = unpack_bf16(rv[e, pl.ds(16 * j, 16)])
                sc_[e, pl.ds(32 * j, 16)] = lo * bc
                sc_[e, pl.ds(32 * j + 16, 16)] = hi * bc
            return carry

        lax.fori_loop(0, CHUNK, edge_body, 0, unroll=2)

    for b in (0, 1):
        unpack_idx(b, b)
        fire_gather(b)
    for i in (0, 1):
        wait_gather(i)
        compute(i, i)
        fire_scatter(i, i)
        unpack_idx(i + 2, i + 2)
        fire_gather(i + 2)

    def macro(g, carry):
        for k in range(4):
            i = 4 * g + 2 + k
            bi = (2 + k) % 4
            b2 = k
            sb = k % 2
            wait_gather(bi)
            wait_scatter(sb)
            compute(bi, sb)
            fire_scatter(sb, bi)
            unpack_idx(b2, i + 2)
            fire_gather(b2)
        return carry

    lax.fori_loop(0, (NCHT - 4) // 4, macro, 0)

    for b in (2, 3):
        sb = b % 2
        wait_gather(b)
        wait_scatter(sb)
        compute(b, sb)
        fire_scatter(sb, b)
    for sb in (0, 1):
        wait_scatter(sb)

    plsc.subcore_barrier()
    pltpu.sync_copy(acc.at[pl.ds(s * RPT, RPT)],
                    out_hbm.at[c, pl.ds(s * RPT, RPT)])


_edge_kernel = functools.partial(
    pl.kernel,
    out_type=jax.ShapeDtypeStruct((NC, N_PAD, AC_W), jnp.float32),
    mesh=plsc.VectorSubcoreMesh(core_axis_name="c", subcore_axis_name="s"),
    compiler_params=pltpu.CompilerParams(use_tc_tiling_on_sc=False),
    scratch_types=[
        pltpu.VMEM((NCHT, CHUNK), jnp.int32),
        [pltpu.VMEM((CHUNK,), jnp.int32) for _ in range(4)],
        [pltpu.VMEM((CHUNK,), jnp.int32) for _ in range(4)],
        [pltpu.VMEM((CHUNK, HS_W // 2), jnp.int32) for _ in range(4)],
        [pltpu.VMEM((CHUNK, AD_W), jnp.float32) for _ in range(4)],
        [pltpu.VMEM((CHUNK, AC_W), jnp.float32) for _ in range(2)],
        pltpu.VMEM_SHARED((N_PAD, AC_W), jnp.float32),
        pltpu.SemaphoreType.DMA,
        [pltpu.SemaphoreType.DMA for _ in range(4)],
        [pltpu.SemaphoreType.DMA for _ in range(2)],
    ],
)(_edge_body)


def kernel(x, W_mu1, a_s_mu1, a_d_mu1, W_ls1, a_s_ls1, a_d_ls1,
           W_mu2, a_s_mu2, a_d_mu2, W_ls2, a_s_ls2, a_d_ls2, edge_index):
    W_all = jnp.concatenate([W_mu1, W_ls1, W_mu2, W_ls2], axis=1)
    amat_s = jnp.zeros((D_IN, 16), jnp.float32)
    amat_d = jnp.zeros((D_IN, AD_W), jnp.float32)
    for l, (a_s, a_d) in enumerate([(a_s_mu1, a_d_mu1), (a_s_ls1, a_d_ls1),
                                    (a_s_mu2, a_d_mu2), (a_s_ls2, a_d_ls2)]):
        amat_s = amat_s.at[32 * l:32 * (l + 1), 2 * l].set(a_s)
        amat_d = amat_d.at[32 * l:32 * (l + 1), l].set(a_d)
    perm = jnp.asarray(_PERM)
    W_perm = W_all[:, perm]
    amat_s = amat_s[perm, :]
    amat_d = amat_d[perm, :]

    hs, ad = pl.pallas_call(
        _proj_body,
        grid=(N // _ROWBLK,),
        in_specs=[
            pl.BlockSpec((_ROWBLK, D_IN), lambda i: (i, 0)),
            pl.BlockSpec((D_IN, D_IN), lambda i: (0, 0)),
            pl.BlockSpec((D_IN, 16), lambda i: (0, 0)),
            pl.BlockSpec((D_IN, AD_W), lambda i: (0, 0)),
        ],
        out_specs=[
            pl.BlockSpec((_ROWBLK, HS_W), lambda i: (i, 0)),
            pl.BlockSpec((_ROWBLK, AD_W), lambda i: (i, 0)),
        ],
        out_shape=[
            jax.ShapeDtypeStruct((N, HS_W), jnp.bfloat16),
            jax.ShapeDtypeStruct((N, AD_W), jnp.float32),
        ],
    )(x, W_perm, amat_s, amat_d)

    ei = edge_index.astype(jnp.int32)
    pad = E_PAD - E
    src = jnp.concatenate([ei[0], jnp.zeros((pad,), jnp.int32)])
    dst = jnp.concatenate([ei[1], jnp.full((pad,), N_PAD - 1, jnp.int32)])
    packed = (jnp.left_shift(src, 14) | dst).reshape(NW, NCHT, CHUNK)

    hs_i32 = lax.bitcast_convert_type(
        hs.reshape(N, HS_W // 2, 2), jnp.int32)
    partials = _edge_kernel(packed, hs_i32, ad)

    kz = jax.random.split(jax.random.key(42), 2)
    n2 = jax.random.normal(kz[0], (N, LAT), jnp.float32)
    n1 = jax.random.normal(kz[1], (N, LAT), jnp.float32)
    noise = jnp.concatenate([n1, n2], axis=1)
    noise_pad = jnp.zeros((N_PAD, 2 * LAT), jnp.float32).at[:N].set(noise)

    z = pl.pallas_call(
        _finalize_body,
        grid=(N_PAD // _FROWBLK,),
        in_specs=[
            pl.BlockSpec((NC, _FROWBLK, AC_W), lambda i: (0, i, 0)),
            pl.BlockSpec((_FROWBLK, 2 * LAT), lambda i: (i, 0)),
        ],
        out_specs=pl.BlockSpec((_FROWBLK, 2 * LAT), lambda i: (i, 0)),
        out_shape=jax.ShapeDtypeStruct((N_PAD, 2 * LAT), jnp.float32),
    )(partials, noise_pad)
    return z[:N]


# speedup vs baseline: 1.1953x; 1.1953x over previous
import functools

import jax
import jax.numpy as jnp
import numpy as np
from jax import lax
from jax.experimental import pallas as pl
from jax.experimental.pallas import tpu as pltpu
from jax.experimental.pallas import tpu_sc as plsc

N = 10000
E = 320000
D_IN = 128
LAT = 32
MAX_LOGSTD = 10.0

HS_W = 160
AC_W = 144
AD_W = 16
NC = 2
NS = 16
NW = NC * NS
CHUNK = 40
NCHT = 256
E_PAD = NW * NCHT * CHUNK
N_PAD = 10240
RPT = N_PAD // NS

_ROWBLK = 1000
_FROWBLK = 1024

_PERM = np.empty(D_IN, np.int32)
for _m in range(64):
    _j, _k = _m // 16, _m % 16
    _PERM[_m] = 32 * _j + _k
    _PERM[64 + _m] = 32 * _j + 16 + _k


def _rne16(v):
    b = lax.bitcast_convert_type(v, jnp.int32)
    return lax.shift_right_logical(
        b + 32767 + jnp.bitwise_and(lax.shift_right_logical(b, 16), 1), 16)


def _proj_body(x_ref, w_ref, amat_s_ref, amat_d_ref, hs_ref, ad_ref):
    h = jnp.dot(x_ref[...], w_ref[...], preferred_element_type=jnp.float32)
    lo = _rne16(h[:, 0:64])
    hi = _rne16(h[:, 64:128])
    hs_ref[:, 0:64] = jnp.bitwise_or(lax.shift_left(hi, 16), lo)
    asr = jnp.dot(h, amat_s_ref[...], preferred_element_type=jnp.float32)
    hs_ref[:, 64:80] = _rne16(asr)
    ad_ref[...] = jnp.dot(h, amat_d_ref[...],
                          preferred_element_type=jnp.float32)


def _finalize_body(p_ref, noise_ref, z_ref):
    a = p_ref[0] + p_ref[1]
    eps = 1e-16
    mu1 = a[:, 0:32] / (a[:, 128:129] + eps)
    ls1 = a[:, 32:64] / (a[:, 129:130] + eps)
    mu2 = a[:, 64:96] / (a[:, 130:131] + eps)
    ls2 = a[:, 96:128] / (a[:, 131:132] + eps)
    z_ref[:, 0:32] = mu1 + noise_ref[:, 0:32] * jnp.exp(
        jnp.minimum(ls1, MAX_LOGSTD))
    z_ref[:, 32:64] = mu2 + noise_ref[:, 32:64] * jnp.exp(
        jnp.minimum(ls2, MAX_LOGSTD))


def _edge_body(pk_hbm, hs_hbm, ad_hbm, out_hbm,
               pk, srcv, dstv, rows, adr, scat, acc, isem, gsem, ssem):
    c = lax.axis_index("c")
    s = lax.axis_index("s")
    wid = s * NC + c

    pltpu.async_copy(pk_hbm.at[wid], pk, isem)

    zb = scat[0]
    for r in range(CHUNK):
        for o in range(0, AC_W, 16):
            zb[r, pl.ds(o, 16)] = jnp.zeros((16,), jnp.float32)
    for t in range(RPT // CHUNK):
        pltpu.sync_copy(zb, acc.at[pl.ds(s * RPT + t * CHUNK, CHUNK)])

    pltpu.make_async_copy(pk_hbm.at[wid], pk, isem).wait()
    plsc.subcore_barrier()

    def unpack_idx(b, j):
        for o in (0, 16, 24):
            v = pk[j, pl.ds(o, 16)]
            srcv[b][pl.ds(o, 16)] = lax.shift_right_logical(v, 14)
            dstv[b][pl.ds(o, 16)] = jnp.bitwise_and(v, 16383)

    def fire_gather(b):
        pltpu.async_copy(hs_hbm.at[srcv[b]], rows[b], gsem[b])
        pltpu.async_copy(ad_hbm.at[dstv[b]], adr[b], gsem[b])

    def wait_gather(b):
        pltpu.make_async_copy(hs_hbm.at[srcv[b]], rows[b], gsem[b]).wait()
        pltpu.make_async_copy(ad_hbm.at[dstv[b]], adr[b], gsem[b]).wait()

    def fire_scatter(sb, b):
        pltpu.async_copy(scat[sb], acc.at[dstv[b]], ssem[sb], add=True)

    def wait_scatter(sb):
        pltpu.make_async_copy(scat[sb], acc.at[dstv[sb]], ssem[sb]).wait()

    def compute(b, sb):
        rv = rows[b]
        ar = adr[b]
        sc_ = scat[sb]

        def unpack_bf16(vi):
            lo = lax.bitcast_convert_type(lax.shift_left(vi, 16),
                                          jnp.float32)
            hi = lax.bitcast_convert_type(
                jnp.bitwise_and(vi, jnp.int32(-65536)), jnp.float32)
            return lo, hi

        def edge_body(e, carry):
            asr, _ = unpack_bf16(rv[e, pl.ds(64, 16)])
            av = asr + ar[e, :]
            ev = jnp.where(av > 0.0, av, av * jnp.float32(0.2))
            exv = jnp.exp(ev)
            sc_[e, pl.ds(D_IN, 16)] = exv
            for j in range(4):
                bc = lax.gather(
                    exv, jnp.full((16, 1), j, jnp.int32),
                    lax.GatherDimensionNumbers(
                        offset_dims=(), collapsed_slice_dims=(0,),
                        start_index_map=(0,)),
                    slice_sizes=(1,),
                    mode=lax.GatherScatterMode.PROMISE_IN_BOUNDS)
                lo, hi = unpack_bf16(rv[e, pl.ds(16 * j, 16)])
                sc_[e, pl.ds(32 * j, 16)] = lo * bc
                sc_[e, pl.ds(32 * j + 16, 16)] = hi * bc
            return carry

        lax.fori_loop(0, CHUNK, edge_body, 0, unroll=2)

    for b in (0, 1):
        unpack_idx(b, b)
        fire_gather(b)
    for i in (0, 1):
        wait_gather(i)
        compute(i, i)
        fire_scatter(i, i)
        unpack_idx(i + 2, i + 2)
        fire_gather(i + 2)

    def macro(g, carry):
        for k in range(4):
            i = 4 * g + 2 + k
            bi = (2 + k) % 4
            b2 = k
            sb = k % 2
            wait_gather(bi)
            wait_scatter(sb)
            compute(bi, sb)
            fire_scatter(sb, bi)
            unpack_idx(b2, i + 2)
            fire_gather(b2)
        return carry

    lax.fori_loop(0, (NCHT - 4) // 4, macro, 0)

    for b in (2, 3):
        sb = b % 2
        wait_gather(b)
        wait_scatter(sb)
        compute(b, sb)
        fire_scatter(sb, b)
    for sb in (0, 1):
        wait_scatter(sb)

    plsc.subcore_barrier()
    pltpu.sync_copy(acc.at[pl.ds(s * RPT, RPT)],
                    out_hbm.at[c, pl.ds(s * RPT, RPT)])


_edge_kernel = functools.partial(
    pl.kernel,
    out_type=jax.ShapeDtypeStruct((NC, N_PAD, AC_W), jnp.float32),
    mesh=plsc.VectorSubcoreMesh(core_axis_name="c", subcore_axis_name="s"),
    compiler_params=pltpu.CompilerParams(use_tc_tiling_on_sc=False),
    scratch_types=[
        pltpu.VMEM((NCHT, CHUNK), jnp.int32),
        [pltpu.VMEM((CHUNK,), jnp.int32) for _ in range(4)],
        [pltpu.VMEM((CHUNK,), jnp.int32) for _ in range(4)],
        [pltpu.VMEM((CHUNK, HS_W // 2), jnp.int32) for _ in range(4)],
        [pltpu.VMEM((CHUNK, AD_W), jnp.float32) for _ in range(4)],
        [pltpu.VMEM((CHUNK, AC_W), jnp.float32) for _ in range(2)],
        pltpu.VMEM_SHARED((N_PAD, AC_W), jnp.float32),
        pltpu.SemaphoreType.DMA,
        [pltpu.SemaphoreType.DMA for _ in range(4)],
        [pltpu.SemaphoreType.DMA for _ in range(2)],
    ],
)(_edge_body)


def kernel(x, W_mu1, a_s_mu1, a_d_mu1, W_ls1, a_s_ls1, a_d_ls1,
           W_mu2, a_s_mu2, a_d_mu2, W_ls2, a_s_ls2, a_d_ls2, edge_index):
    W_all = jnp.concatenate([W_mu1, W_ls1, W_mu2, W_ls2], axis=1)
    amat_s = jnp.zeros((D_IN, 16), jnp.float32)
    amat_d = jnp.zeros((D_IN, AD_W), jnp.float32)
    for l, (a_s, a_d) in enumerate([(a_s_mu1, a_d_mu1), (a_s_ls1, a_d_ls1),
                                    (a_s_mu2, a_d_mu2), (a_s_ls2, a_d_ls2)]):
        amat_s = amat_s.at[32 * l:32 * (l + 1), l].set(a_s)
        amat_d = amat_d.at[32 * l:32 * (l + 1), l].set(a_d)
    perm = jnp.asarray(_PERM)
    W_perm = W_all[:, perm]
    amat_s = amat_s[perm, :]
    amat_d = amat_d[perm, :]

    hs, ad = pl.pallas_call(
        _proj_body,
        grid=(N // _ROWBLK,),
        in_specs=[
            pl.BlockSpec((_ROWBLK, D_IN), lambda i: (i, 0)),
            pl.BlockSpec((D_IN, D_IN), lambda i: (0, 0)),
            pl.BlockSpec((D_IN, 16), lambda i: (0, 0)),
            pl.BlockSpec((D_IN, AD_W), lambda i: (0, 0)),
        ],
        out_specs=[
            pl.BlockSpec((_ROWBLK, HS_W // 2), lambda i: (i, 0)),
            pl.BlockSpec((_ROWBLK, AD_W), lambda i: (i, 0)),
        ],
        out_shape=[
            jax.ShapeDtypeStruct((N, HS_W // 2), jnp.int32),
            jax.ShapeDtypeStruct((N, AD_W), jnp.float32),
        ],
    )(x, W_perm, amat_s, amat_d)

    ei = edge_index.astype(jnp.int32)
    pad = E_PAD - E
    src = jnp.concatenate([ei[0], jnp.zeros((pad,), jnp.int32)])
    dst = jnp.concatenate([ei[1], jnp.full((pad,), N_PAD - 1, jnp.int32)])
    packed = (jnp.left_shift(src, 14) | dst).reshape(NW, NCHT, CHUNK)

    partials = _edge_kernel(packed, hs, ad)

    kz = jax.random.split(jax.random.key(42), 2)
    n2 = jax.random.normal(kz[0], (N, LAT), jnp.float32)
    n1 = jax.random.normal(kz[1], (N, LAT), jnp.float32)
    noise = jnp.concatenate([n1, n2], axis=1)
    noise_pad = jnp.zeros((N_PAD, 2 * LAT), jnp.float32).at[:N].set(noise)

    z = pl.pallas_call(
        _finalize_body,
        grid=(N_PAD // _FROWBLK,),
        in_specs=[
            pl.BlockSpec((NC, _FROWBLK, AC_W), lambda i: (0, i, 0)),
            pl.BlockSpec((_FROWBLK, 2 * LAT), lambda i: (i, 0)),
        ],
        out_specs=pl.BlockSpec((_FROWBLK, 2 * LAT), lambda i: (i, 0)),
        out_shape=jax.ShapeDtypeStruct((N_PAD, 2 * LAT), jnp.float32),
    )(partials, noise_pad)
    return z[:N]


# speedup vs baseline: 1.2849x; 1.0749x over previous
import functools

import jax
import jax.numpy as jnp
import numpy as np
from jax import lax
from jax.experimental import pallas as pl
from jax.experimental.pallas import tpu as pltpu
from jax.experimental.pallas import tpu_sc as plsc

N = 10000
E = 320000
D_IN = 128
LAT = 32
MAX_LOGSTD = 10.0

HS_W = 160
AC_W = 144
AD_W = 16
NC = 2
NS = 16
NW = NC * NS
CHUNK = 40
NCHT = 256
E_PAD = NW * NCHT * CHUNK
N_PAD = 10240
RPT = N_PAD // NS

_ROWBLK = 1000
_FROWBLK = 1024

_PERM = np.empty(D_IN, np.int32)
for _m in range(64):
    _j, _k = _m // 16, _m % 16
    _PERM[_m] = 32 * _j + _k
    _PERM[64 + _m] = 32 * _j + 16 + _k


def _rne16(v):
    b = lax.bitcast_convert_type(v, jnp.int32)
    return lax.shift_right_logical(
        b + 32767 + jnp.bitwise_and(lax.shift_right_logical(b, 16), 1), 16)


def _proj_body(x_ref, w_ref, amat_s_ref, amat_d_ref, hs_ref, ad_ref):
    h = jnp.dot(x_ref[...], w_ref[...], preferred_element_type=jnp.float32)
    lo = _rne16(h[:, 0:64])
    hi = _rne16(h[:, 64:128])
    hs_ref[:, 0:64] = jnp.bitwise_or(lax.shift_left(hi, 16), lo)
    asr = jnp.dot(h, amat_s_ref[...], preferred_element_type=jnp.float32)
    hs_ref[:, 64:80] = _rne16(asr)
    ad_ref[...] = jnp.dot(h, amat_d_ref[...],
                          preferred_element_type=jnp.float32)


def _finalize_body(p_ref, noise_ref, z_ref):
    a = p_ref[0] + p_ref[1]
    eps = 1e-16
    mu1 = a[:, 0:32] / (a[:, 128:129] + eps)
    ls1 = a[:, 32:64] / (a[:, 129:130] + eps)
    mu2 = a[:, 64:96] / (a[:, 130:131] + eps)
    ls2 = a[:, 96:128] / (a[:, 131:132] + eps)
    z_ref[:, 0:32] = mu1 + noise_ref[:, 0:32] * jnp.exp(
        jnp.minimum(ls1, MAX_LOGSTD))
    z_ref[:, 32:64] = mu2 + noise_ref[:, 32:64] * jnp.exp(
        jnp.minimum(ls2, MAX_LOGSTD))


def _edge_body(pk_hbm, hs_hbm, ad_hbm, out_hbm,
               pk, srcv, dstv, sdst, rows, adr, scat, acc,
               isem, gsem, ssem):
    c = lax.axis_index("c")
    s = lax.axis_index("s")
    wid = s * NC + c

    pltpu.async_copy(pk_hbm.at[wid], pk, isem)

    zb = scat[0]
    for r in range(CHUNK):
        for o in range(0, AC_W, 16):
            zb[r, pl.ds(o, 16)] = jnp.zeros((16,), jnp.float32)
    for t in range(RPT // CHUNK):
        pltpu.sync_copy(zb, acc.at[pl.ds(s * RPT + t * CHUNK, CHUNK)])

    pltpu.make_async_copy(pk_hbm.at[wid], pk, isem).wait()
    plsc.subcore_barrier()

    def unpack_idx(b, j):
        for o in (0, 16, 24):
            v = pk[j, pl.ds(o, 16)]
            srcv[b][pl.ds(o, 16)] = lax.shift_right_logical(v, 14)
            dstv[b][pl.ds(o, 16)] = jnp.bitwise_and(v, 16383)

    def fire_gather(b):
        pltpu.async_copy(hs_hbm.at[srcv[b]], rows[b], gsem[b])
        pltpu.async_copy(ad_hbm.at[dstv[b]], adr[b], gsem[b])

    def wait_gather(b):
        pltpu.make_async_copy(hs_hbm.at[srcv[b]], rows[b], gsem[b]).wait()
        pltpu.make_async_copy(ad_hbm.at[dstv[b]], adr[b], gsem[b]).wait()

    def copy_dst(b, sb):
        for o in (0, 16, 24):
            sdst[sb][pl.ds(o, 16)] = dstv[b][pl.ds(o, 16)]

    def fire_scatter(sb):
        pltpu.async_copy(scat[sb], acc.at[sdst[sb]], ssem[sb], add=True)

    def wait_scatter(sb):
        pltpu.make_async_copy(scat[sb], acc.at[sdst[sb]], ssem[sb]).wait()

    def compute(b, sb):
        rv = rows[b]
        ar = adr[b]
        sc_ = scat[sb]

        def unpack_bf16(vi):
            lo = lax.bitcast_convert_type(lax.shift_left(vi, 16),
                                          jnp.float32)
            hi = lax.bitcast_convert_type(
                jnp.bitwise_and(vi, jnp.int32(-65536)), jnp.float32)
            return lo, hi

        def edge_body(e, carry):
            asr, _ = unpack_bf16(rv[e, pl.ds(64, 16)])
            av = asr + ar[e, :]
            ev = jnp.where(av > 0.0, av, av * jnp.float32(0.2))
            exv = jnp.exp(ev)
            sc_[e, pl.ds(D_IN, 16)] = exv
            for j in range(4):
                bc = lax.gather(
                    exv, jnp.full((16, 1), j, jnp.int32),
                    lax.GatherDimensionNumbers(
                        offset_dims=(), collapsed_slice_dims=(0,),
                        start_index_map=(0,)),
                    slice_sizes=(1,),
                    mode=lax.GatherScatterMode.PROMISE_IN_BOUNDS)
                lo, hi = unpack_bf16(rv[e, pl.ds(16 * j, 16)])
                sc_[e, pl.ds(32 * j, 16)] = lo * bc
                sc_[e, pl.ds(32 * j + 16, 16)] = hi * bc
            return carry

        lax.fori_loop(0, CHUNK, edge_body, 0, unroll=2)

    for b in range(4):
        unpack_idx(b, b)
        fire_gather(b)
    for i in (0, 1, 2, 3):
        sb = i % 2
        wait_gather(i)
        if i >= 2:
            wait_scatter(sb)
        copy_dst(i, sb)
        compute(i, sb)
        fire_scatter(sb)
        unpack_idx(i, i + 4)
        fire_gather(i)

    def macro(g, carry):
        for k in range(4):
            i = 4 * g + k
            sb = k % 2
            wait_gather(k)
            wait_scatter(sb)
            copy_dst(k, sb)
            compute(k, sb)
            fire_scatter(sb)
            unpack_idx(k, i + 4)
            fire_gather(k)
        return carry

    lax.fori_loop(1, NCHT // 4 - 1, macro, 0)

    for k in range(4):
        sb = k % 2
        wait_gather(k)
        wait_scatter(sb)
        copy_dst(k, sb)
        compute(k, sb)
        fire_scatter(sb)
    for sb in (0, 1):
        wait_scatter(sb)

    plsc.subcore_barrier()
    pltpu.sync_copy(acc.at[pl.ds(s * RPT, RPT)],
                    out_hbm.at[c, pl.ds(s * RPT, RPT)])


_edge_kernel = functools.partial(
    pl.kernel,
    out_type=jax.ShapeDtypeStruct((NC, N_PAD, AC_W), jnp.float32),
    mesh=plsc.VectorSubcoreMesh(core_axis_name="c", subcore_axis_name="s"),
    compiler_params=pltpu.CompilerParams(use_tc_tiling_on_sc=False),
    scratch_types=[
        pltpu.VMEM((NCHT, CHUNK), jnp.int32),
        [pltpu.VMEM((CHUNK,), jnp.int32) for _ in range(4)],
        [pltpu.VMEM((CHUNK,), jnp.int32) for _ in range(4)],
        [pltpu.VMEM((CHUNK,), jnp.int32) for _ in range(2)],
        [pltpu.VMEM((CHUNK, HS_W // 2), jnp.int32) for _ in range(4)],
        [pltpu.VMEM((CHUNK, AD_W), jnp.float32) for _ in range(4)],
        [pltpu.VMEM((CHUNK, AC_W), jnp.float32) for _ in range(2)],
        pltpu.VMEM_SHARED((N_PAD, AC_W), jnp.float32),
        pltpu.SemaphoreType.DMA,
        [pltpu.SemaphoreType.DMA for _ in range(4)],
        [pltpu.SemaphoreType.DMA for _ in range(2)],
    ],
)(_edge_body)


def kernel(x, W_mu1, a_s_mu1, a_d_mu1, W_ls1, a_s_ls1, a_d_ls1,
           W_mu2, a_s_mu2, a_d_mu2, W_ls2, a_s_ls2, a_d_ls2, edge_index):
    W_all = jnp.concatenate([W_mu1, W_ls1, W_mu2, W_ls2], axis=1)
    amat_s = jnp.zeros((D_IN, 16), jnp.float32)
    amat_d = jnp.zeros((D_IN, AD_W), jnp.float32)
    for l, (a_s, a_d) in enumerate([(a_s_mu1, a_d_mu1), (a_s_ls1, a_d_ls1),
                                    (a_s_mu2, a_d_mu2), (a_s_ls2, a_d_ls2)]):
        amat_s = amat_s.at[32 * l:32 * (l + 1), l].set(a_s)
        amat_d = amat_d.at[32 * l:32 * (l + 1), l].set(a_d)
    perm = jnp.asarray(_PERM)
    W_perm = W_all[:, perm]
    amat_s = amat_s[perm, :]
    amat_d = amat_d[perm, :]

    hs, ad = pl.pallas_call(
        _proj_body,
        grid=(N // _ROWBLK,),
        in_specs=[
            pl.BlockSpec((_ROWBLK, D_IN), lambda i: (i, 0)),
            pl.BlockSpec((D_IN, D_IN), lambda i: (0, 0)),
            pl.BlockSpec((D_IN, 16), lambda i: (0, 0)),
            pl.BlockSpec((D_IN, AD_W), lambda i: (0, 0)),
        ],
        out_specs=[
            pl.BlockSpec((_ROWBLK, HS_W // 2), lambda i: (i, 0)),
            pl.BlockSpec((_ROWBLK, AD_W), lambda i: (i, 0)),
        ],
        out_shape=[
            jax.ShapeDtypeStruct((N, HS_W // 2), jnp.int32),
            jax.ShapeDtypeStruct((N, AD_W), jnp.float32),
        ],
    )(x, W_perm, amat_s, amat_d)

    ei = edge_index.astype(jnp.int32)
    pad = E_PAD - E
    src = jnp.concatenate([ei[0], jnp.zeros((pad,), jnp.int32)])
    dst = jnp.concatenate([ei[1], jnp.full((pad,), N_PAD - 1, jnp.int32)])
    packed = (jnp.left_shift(src, 14) | dst).reshape(NW, NCHT, CHUNK)

    partials = _edge_kernel(packed, hs, ad)

    kz = jax.random.split(jax.random.key(42), 2)
    n2 = jax.random.normal(kz[0], (N, LAT), jnp.float32)
    n1 = jax.random.normal(kz[1], (N, LAT), jnp.float32)
    noise = jnp.concatenate([n1, n2], axis=1)
    noise_pad = jnp.zeros((N_PAD, 2 * LAT), jnp.float32).at[:N].set(noise)

    z = pl.pallas_call(
        _finalize_body,
        grid=(N_PAD // _FROWBLK,),
        in_specs=[
            pl.BlockSpec((NC, _FROWBLK, AC_W), lambda i: (0, i, 0)),
            pl.BlockSpec((_FROWBLK, 2 * LAT), lambda i: (i, 0)),
        ],
        out_specs=pl.BlockSpec((_FROWBLK, 2 * LAT), lambda i: (i, 0)),
        out_shape=jax.ShapeDtypeStruct((N_PAD, 2 * LAT), jnp.float32),
    )(partials, noise_pad)
    return z[:N]
